# baseline (speedup 1.0000x reference)
import jax
import jax.numpy as jnp
import numpy as np
from jax import lax
from jax.experimental import pallas as pl
from jax.experimental.pallas import tpu as pltpu
from jax.experimental.pallas import tpu_sc as plsc

N_V = 100000
N_VP = 100096
N_F = 1600000
NC = 2
NS = 16
NW = NC * NS
FACES_PER_W = N_F // NW
CHUNK_F = 80
CHUNK_S = CHUNK_F * 3
N_CHUNKS = FACES_PER_W // CHUNK_F
ROWS_PER_TILE = N_VP // NS


def _face_pass_body(verts_hbm, idx_hbm, zeros_hbm, out_hbm, acc,
                    ichkA, sidxA, exA, eyA, ezA, vxA, vyA, vzA, ubufA,
                    ichkB, sidxB, exB, eyB, ezB, vxB, vyB, vzB, ubufB,
                    isemA, gsemA, ssemA, isemB, gsemB, ssemB):
    cid = lax.axis_index("c")
    sid = lax.axis_index("s")
    wid = sid * NC + cid

    pltpu.sync_copy(zeros_hbm.at[pl.ds(sid * ROWS_PER_TILE, ROWS_PER_TILE)],
                    acc.at[pl.ds(sid * ROWS_PER_TILE, ROWS_PER_TILE)])
    plsc.subcore_barrier()

    lane = lax.iota(jnp.int32, 16)
    rot_idx = {sh: ((lane + sh) % 16, (lane + (16 - sh)) % 16)
               for sh in (1, 2, 4, 8)}
    stage_keep = {sh: (lane & sh) == 0 for sh in (1, 2, 4, 8)}
    keep_a = (lane < 9) | (lane == 15)
    keep_b = (lane < 6) | ((lane >= 9) & (lane < 12)) | (lane == 15)
    keep_c = (lane < 6) | (lane >= 12)
    zero16 = jnp.zeros((16,), jnp.float32)
    ones16 = jnp.ones((16,), jnp.float32)

    bufA = (ichkA, sidxA, exA, eyA, ezA, vxA, vyA, vzA, ubufA, isemA, gsemA, ssemA)
    bufB = (ichkB, sidxB, exB, eyB, ezB, vxB, vyB, vzB, ubufB, isemB, gsemB, ssemB)

    def transpose16(v):
        for sh in (8, 4, 2, 1):
            fwd, bwd = rot_idx[sh]
            keep = stage_keep[sh]
            nv = list(v)
            for i in range(16):
                if i & sh == 0:
                    j = i + sh
                    nv[i] = jnp.where(keep, v[i], v[j][bwd])
                    nv[j] = jnp.where(keep, v[i][fwd], v[j])
            v = nv
        return v

    def fetch(t, bp):
        ichk, sidx, ex, ey, ez, vx, vy, vz, ubuf, isem, gsem, ssem = bp
        fbase = wid * FACES_PER_W + t * CHUNK_F
        ds = [pltpu.async_copy(idx_hbm.at[pl.ds(s * N_F + fbase, CHUNK_F)],
                               sidx.at[pl.ds(s * CHUNK_F, CHUNK_F)], isem)
              for s in range(3)]
        for d in ds:
            d.wait()

        def expand_body(j, _):
            v = sidx[pl.ds(j * 16, 16)]
            e = v * 3
            ex[pl.ds(j * 16, 16)] = e
            ey[pl.ds(j * 16, 16)] = e + 1
            ez[pl.ds(j * 16, 16)] = e + 2
            return ()

        lax.fori_loop(0, CHUNK_S // 16, expand_body, ())
        pltpu.async_copy(verts_hbm.at[ex], vx, gsem)
        pltpu.async_copy(verts_hbm.at[ey], vy, gsem)
        pltpu.async_copy(verts_hbm.at[ez], vz, gsem)

    def compute_scatter(bp):
        ichk, sidx, ex, ey, ez, vx, vy, vz, ubuf, isem, gsem, ssem = bp
        pltpu.make_async_copy(verts_hbm.at[ex], vx, gsem).wait()
        pltpu.make_async_copy(verts_hbm.at[ey], vy, gsem).wait()
        pltpu.make_async_copy(verts_hbm.at[ez], vz, gsem).wait()

        def group_body(j, _):
            oa = j * 16
            ob = CHUNK_F + j * 16
            oc = 2 * CHUNK_F + j * 16
            ax = vx[pl.ds(oa, 16)]
            ay = vy[pl.ds(oa, 16)]
            az = vz[pl.ds(oa, 16)]
            bx = vx[pl.ds(ob, 16)]
            by = vy[pl.ds(ob, 16)]
            bz = vz[pl.ds(ob, 16)]
            cx = vx[pl.ds(oc, 16)]
            cy = vy[pl.ds(oc, 16)]
            cz = vz[pl.ds(oc, 16)]

            e1x, e1y, e1z = bx - ax, by - ay, bz - az
            e2x, e2y, e2z = cx - ax, cy - ay, cz - az
            wx = e1y * e2z - e1z * e2y
            wy = e1z * e2x - e1x * e2z
            wz = e1x * e2y - e1y * e2x
            n2 = jnp.maximum(wx * wx + wy * wy + wz * wz, 1e-20)

            i32 = lax.bitcast_convert_type(n2, jnp.int32)
            i32 = 0x5F3759DF - (i32 >> 1)
            y = lax.bitcast_convert_type(i32, jnp.float32)
            h = 0.5 * n2
            y = y * (1.5 - h * y * y)
            y = y * (1.5 - h * y * y)
            y = y * (1.5 - h * y * y)

            rows = transpose16([
                wx * y, wy * y, wz * y,
                wx, wy, wz,
                e2x, e2y, e2z,
                -e1x, -e1y, -e1z,
                e1x - e2x, e1y - e2y, e1z - e2z,
                ones16,
            ])
            for i in range(16):
                r = rows[i]
                ubuf[oa + i] = jnp.where(keep_a, r, zero16)
                ubuf[ob + i] = jnp.where(keep_b, r, zero16)
                ubuf[oc + i] = jnp.where(keep_c, r, zero16)
            return ()

        lax.fori_loop(0, CHUNK_F // 16, group_body, ())
        pltpu.async_copy(ubuf, acc.at[sidx], ssem, add=True)

    def wait_scatter(bp):
        ichk, sidx, ex, ey, ez, vx, vy, vz, ubuf, isem, gsem, ssem = bp
        pltpu.make_async_copy(ubuf, acc.at[sidx], ssem).wait()

    fetch(0, bufA)
    fetch(1, bufB)
    compute_scatter(bufA)

    def pair_body(u, _):
        t0 = 2 * u + 1
        wait_scatter(bufA)
        fetch(t0 + 1, bufA)
        compute_scatter(bufB)
        wait_scatter(bufB)

        @pl.when(u < (N_CHUNKS - 3) // 2)
        def _():
            fetch(t0 + 2, bufB)

        compute_scatter(bufA)
        return ()

    lax.fori_loop(0, (N_CHUNKS - 1) // 2, pair_body, ())
    wait_scatter(bufA)
    plsc.subcore_barrier()

    pltpu.sync_copy(acc.at[pl.ds(sid * ROWS_PER_TILE, ROWS_PER_TILE)],
                    out_hbm.at[cid, pl.ds(sid * ROWS_PER_TILE, ROWS_PER_TILE)])


_face_pass = pl.kernel(
    _face_pass_body,
    out_type=jax.ShapeDtypeStruct((NC, N_VP, 16), jnp.float32),
    mesh=plsc.VectorSubcoreMesh(core_axis_name="c", subcore_axis_name="s",
                                num_cores=NC, num_subcores=NS),
    scratch_types=[
        pltpu.VMEM_SHARED((N_VP, 16), jnp.float32),
    ] + 2 * [
        pltpu.VMEM((CHUNK_S,), jnp.int32),
        pltpu.VMEM((CHUNK_S,), jnp.int32),
        pltpu.VMEM((CHUNK_S,), jnp.int32),
        pltpu.VMEM((CHUNK_S,), jnp.int32),
        pltpu.VMEM((CHUNK_S,), jnp.int32),
        pltpu.VMEM((CHUNK_S,), jnp.float32),
        pltpu.VMEM((CHUNK_S,), jnp.float32),
        pltpu.VMEM((CHUNK_S,), jnp.float32),
        pltpu.VMEM((CHUNK_S, 16), jnp.float32),
    ] + 6 * [pltpu.SemaphoreType.DMA],
    compiler_params=pltpu.CompilerParams(use_tc_tiling_on_sc=False),
)

BS = 1000


def _finalize_body(part_ref, verts_ref, out_ref):
    p = part_ref[...]
    a = p[0] + p[1]
    nsum = a[:, 0:3]
    wsum = a[:, 3:6]
    lap = a[:, 6:9] + a[:, 9:12] + a[:, 12:15]
    cnt = a[:, 15:16]

    normals = nsum * lax.rsqrt(
        jnp.clip(jnp.sum(nsum * nsum, -1, keepdims=True), 1e-20, None))
    va = jnp.sum(wsum * normals, -1, keepdims=True)
    areas = jnp.clip(va, 1e-10, None) / 6.0
    lsa = jnp.log(areas / 2.0) * 0.5

    z = jnp.concatenate(
        [jnp.zeros((BS, 2), jnp.float32), jnp.ones((BS, 1), jnp.float32)], -1)
    ul = jnp.where(cnt > 0, lap / jnp.maximum(cnt, 1.0), z)
    means = verts_ref[...][:, 0:3] - ul * 0.5

    w = 1.0 + normals[:, 2:3]
    q = jnp.concatenate(
        [w, -normals[:, 1:2], normals[:, 0:1], jnp.zeros_like(w)], -1)
    quats = q * lax.rsqrt(
        jnp.clip(jnp.sum(q * q, -1, keepdims=True), 1e-20, None))

    scales = jnp.concatenate(
        [lsa, lsa, jnp.full_like(lsa, float(np.log(1e-10)))], -1)
    out_ref[...] = jnp.concatenate([means, normals, scales, quats], -1)


_finalize = pl.pallas_call(
    _finalize_body,
    grid=(N_V // BS,),
    in_specs=[
        pl.BlockSpec((NC, BS, 16), lambda i: (0, i, 0)),
        pl.BlockSpec((BS, 4), lambda i: (i, 0)),
    ],
    out_specs=pl.BlockSpec((BS, 13), lambda i: (i, 0)),
    out_shape=jax.ShapeDtypeStruct((N_V, 13), jnp.float32),
)


@jax.jit
def kernel(vertices, indices):
    verts_flat = vertices.reshape(-1)
    idx_flat = indices.astype(jnp.int32).reshape(-1)
    zeros = jnp.zeros((N_VP, 16), jnp.float32)
    verts_p = jnp.pad(vertices, ((0, 0), (0, 1)))
    partials = _face_pass(verts_flat, idx_flat, zeros)
    return _finalize(partials, verts_p)


# speedup vs baseline: 4.0756x; 4.0756x over previous
import jax
import jax.numpy as jnp
import numpy as np
from jax import lax
from jax.experimental import pallas as pl
from jax.experimental.pallas import tpu as pltpu
from jax.experimental.pallas import tpu_sc as plsc

N_V = 100000
N_VP = 100096
N_F = 1600000
NC = 2
NS = 16
NW = NC * NS
FACES_PER_W = N_F // NW
CHUNK_F = 80
CHUNK_S = CHUNK_F * 3
N_CHUNKS = FACES_PER_W // CHUNK_F
ROWS_PER_TILE = N_VP // NS


def _face_pass_body(verts_hbm, idx_hbm, zeros_hbm, out_hbm, acc,
                    ichkA, sidxA, exA, eyA, ezA, vxA, vyA, vzA, ubufA,
                    ichkB, sidxB, exB, eyB, ezB, vxB, vyB, vzB, ubufB,
                    isemA, gsemA, ssemA, isemB, gsemB, ssemB):
    cid = lax.axis_index("c")
    sid = lax.axis_index("s")
    wid = sid * NC + cid

    pltpu.sync_copy(zeros_hbm.at[pl.ds(sid * ROWS_PER_TILE, ROWS_PER_TILE)],
                    acc.at[pl.ds(sid * ROWS_PER_TILE, ROWS_PER_TILE)])
    plsc.subcore_barrier()

    lane = lax.iota(jnp.int32, 16)
    rot_idx = {sh: ((lane + sh) % 16, (lane + (16 - sh)) % 16)
               for sh in (1, 2, 4, 8)}
    stage_keep = {sh: (lane & sh) == 0 for sh in (1, 2, 4, 8)}
    keep_a = (lane < 9) | (lane == 15)
    keep_b = (lane < 6) | ((lane >= 9) & (lane < 12)) | (lane == 15)
    keep_c = (lane < 6) | (lane >= 12)
    zero16 = jnp.zeros((16,), jnp.float32)
    ones16 = jnp.ones((16,), jnp.float32)

    bufA = (ichkA, sidxA, exA, eyA, ezA, vxA, vyA, vzA, ubufA, isemA, gsemA, ssemA)
    bufB = (ichkB, sidxB, exB, eyB, ezB, vxB, vyB, vzB, ubufB, isemB, gsemB, ssemB)

    def transpose16(v):
        for sh in (8, 4, 2, 1):
            fwd, bwd = rot_idx[sh]
            keep = stage_keep[sh]
            nv = list(v)
            for i in range(16):
                if i & sh == 0:
                    j = i + sh
                    nv[i] = jnp.where(keep, v[i], v[j][bwd])
                    nv[j] = jnp.where(keep, v[i][fwd], v[j])
            v = nv
        return v

    def fetch(t, bp):
        ichk, sidx, ex, ey, ez, vx, vy, vz, ubuf, isem, gsem, ssem = bp
        fbase = wid * FACES_PER_W + t * CHUNK_F
        ds = [pltpu.async_copy(idx_hbm.at[pl.ds(s * N_F + fbase, CHUNK_F)],
                               sidx.at[pl.ds(s * CHUNK_F, CHUNK_F)], isem)
              for s in range(3)]
        for d in ds:
            d.wait()

        def expand_body(j, _):
            v = sidx[pl.ds(j * 16, 16)]
            ey[pl.ds(j * 16, 16)] = v + N_V
            ez[pl.ds(j * 16, 16)] = v + 2 * N_V
            return ()

        lax.fori_loop(0, CHUNK_S // 16, expand_body, ())
        pltpu.async_copy(verts_hbm.at[sidx], vx, gsem)
        pltpu.async_copy(verts_hbm.at[ey], vy, gsem)
        pltpu.async_copy(verts_hbm.at[ez], vz, gsem)

    def compute_scatter(bp):
        ichk, sidx, ex, ey, ez, vx, vy, vz, ubuf, isem, gsem, ssem = bp
        pltpu.make_async_copy(verts_hbm.at[sidx], vx, gsem).wait()
        pltpu.make_async_copy(verts_hbm.at[ey], vy, gsem).wait()
        pltpu.make_async_copy(verts_hbm.at[ez], vz, gsem).wait()

        def group_body(j, _):
            oa = j * 16
            ob = CHUNK_F + j * 16
            oc = 2 * CHUNK_F + j * 16
            ax = vx[pl.ds(oa, 16)]
            ay = vy[pl.ds(oa, 16)]
            az = vz[pl.ds(oa, 16)]
            bx = vx[pl.ds(ob, 16)]
            by = vy[pl.ds(ob, 16)]
            bz = vz[pl.ds(ob, 16)]
            cx = vx[pl.ds(oc, 16)]
            cy = vy[pl.ds(oc, 16)]
            cz = vz[pl.ds(oc, 16)]

            e1x, e1y, e1z = bx - ax, by - ay, bz - az
            e2x, e2y, e2z = cx - ax, cy - ay, cz - az
            wx = e1y * e2z - e1z * e2y
            wy = e1z * e2x - e1x * e2z
            wz = e1x * e2y - e1y * e2x
            n2 = jnp.maximum(wx * wx + wy * wy + wz * wz, 1e-20)

            i32 = lax.bitcast_convert_type(n2, jnp.int32)
            i32 = 0x5F3759DF - (i32 >> 1)
            y = lax.bitcast_convert_type(i32, jnp.float32)
            h = 0.5 * n2
            y = y * (1.5 - h * y * y)
            y = y * (1.5 - h * y * y)
            y = y * (1.5 - h * y * y)

            rows = transpose16([
                wx * y, wy * y, wz * y,
                wx, wy, wz,
                e2x, e2y, e2z,
                -e1x, -e1y, -e1z,
                e1x - e2x, e1y - e2y, e1z - e2z,
                ones16,
            ])
            for i in range(16):
                r = rows[i]
                ubuf[oa + i] = jnp.where(keep_a, r, zero16)
                ubuf[ob + i] = jnp.where(keep_b, r, zero16)
                ubuf[oc + i] = jnp.where(keep_c, r, zero16)
            return ()

        lax.fori_loop(0, CHUNK_F // 16, group_body, ())
        pltpu.async_copy(ubuf, acc.at[sidx], ssem, add=True)

    def wait_scatter(bp):
        ichk, sidx, ex, ey, ez, vx, vy, vz, ubuf, isem, gsem, ssem = bp
        pltpu.make_async_copy(ubuf, acc.at[sidx], ssem).wait()

    fetch(0, bufA)
    fetch(1, bufB)
    compute_scatter(bufA)

    def pair_body(u, _):
        t0 = 2 * u + 1
        wait_scatter(bufA)
        fetch(t0 + 1, bufA)
        compute_scatter(bufB)
        wait_scatter(bufB)

        @pl.when(u < (N_CHUNKS - 3) // 2)
        def _():
            fetch(t0 + 2, bufB)

        compute_scatter(bufA)
        return ()

    lax.fori_loop(0, (N_CHUNKS - 1) // 2, pair_body, ())
    wait_scatter(bufA)
    plsc.subcore_barrier()

    pltpu.sync_copy(acc.at[pl.ds(sid * ROWS_PER_TILE, ROWS_PER_TILE)],
                    out_hbm.at[cid, pl.ds(sid * ROWS_PER_TILE, ROWS_PER_TILE)])


_face_pass = pl.kernel(
    _face_pass_body,
    out_type=jax.ShapeDtypeStruct((NC, N_VP, 16), jnp.float32),
    mesh=plsc.VectorSubcoreMesh(core_axis_name="c", subcore_axis_name="s",
                                num_cores=NC, num_subcores=NS),
    scratch_types=[
        pltpu.VMEM_SHARED((N_VP, 16), jnp.float32),
    ] + 2 * [
        pltpu.VMEM((CHUNK_S,), jnp.int32),
        pltpu.VMEM((CHUNK_S,), jnp.int32),
        pltpu.VMEM((CHUNK_S,), jnp.int32),
        pltpu.VMEM((CHUNK_S,), jnp.int32),
        pltpu.VMEM((CHUNK_S,), jnp.int32),
        pltpu.VMEM((CHUNK_S,), jnp.float32),
        pltpu.VMEM((CHUNK_S,), jnp.float32),
        pltpu.VMEM((CHUNK_S,), jnp.float32),
        pltpu.VMEM((CHUNK_S, 16), jnp.float32),
    ] + 6 * [pltpu.SemaphoreType.DMA],
    compiler_params=pltpu.CompilerParams(use_tc_tiling_on_sc=False),
)

BS = 1000


def _finalize_body(part_ref, verts_ref, out_ref):
    p = part_ref[...]
    a = p[0] + p[1]
    nsum = a[:, 0:3]
    wsum = a[:, 3:6]
    lap = a[:, 6:9] + a[:, 9:12] + a[:, 12:15]
    cnt = a[:, 15:16]

    normals = nsum * lax.rsqrt(
        jnp.clip(jnp.sum(nsum * nsum, -1, keepdims=True), 1e-20, None))
    va = jnp.sum(wsum * normals, -1, keepdims=True)
    areas = jnp.clip(va, 1e-10, None) / 6.0
    lsa = jnp.log(areas / 2.0) * 0.5

    z = jnp.concatenate(
        [jnp.zeros((BS, 2), jnp.float32), jnp.ones((BS, 1), jnp.float32)], -1)
    ul = jnp.where(cnt > 0, lap / jnp.maximum(cnt, 1.0), z)
    means = verts_ref[...][:, 0:3] - ul * 0.5

    w = 1.0 + normals[:, 2:3]
    q = jnp.concatenate(
        [w, -normals[:, 1:2], normals[:, 0:1], jnp.zeros_like(w)], -1)
    quats = q * lax.rsqrt(
        jnp.clip(jnp.sum(q * q, -1, keepdims=True), 1e-20, None))

    scales = jnp.concatenate(
        [lsa, lsa, jnp.full_like(lsa, float(np.log(1e-10)))], -1)
    out_ref[...] = jnp.concatenate([means, normals, scales, quats], -1)


_finalize = pl.pallas_call(
    _finalize_body,
    grid=(N_V // BS,),
    in_specs=[
        pl.BlockSpec((NC, BS, 16), lambda i: (0, i, 0)),
        pl.BlockSpec((BS, 4), lambda i: (i, 0)),
    ],
    out_specs=pl.BlockSpec((BS, 13), lambda i: (i, 0)),
    out_shape=jax.ShapeDtypeStruct((N_V, 13), jnp.float32),
)


@jax.jit
def kernel(vertices, indices):
    verts_flat = vertices.T.reshape(-1)
    idx_flat = indices.astype(jnp.int32).T.reshape(-1)
    zeros = jnp.zeros((N_VP, 16), jnp.float32)
    verts_p = jnp.pad(vertices, ((0, 0), (0, 1)))
    partials = _face_pass(verts_flat, idx_flat, zeros)
    return _finalize(partials, verts_p)
